# baseline (speedup 1.0000x reference)
import functools

import jax
import jax.numpy as jnp
from jax import lax
from jax.experimental import pallas as pl
from jax.experimental.pallas import tpu as pltpu
from jax.experimental.pallas import tpu_sc as plsc

BATCH = 16384
F = 32
NC = 2
NS = 16
NW = NC * NS
BPW = BATCH // NW
L = 16
GROUPS = BPW // L


def _mf_body(user_hbm, item_hbm, uf_hbm, if_hbm, ub_hbm, ib_hbm, out_hbm,
             uidx_v, iidx_v, uf_v, if_v, ub_v, ib_v, out_v,
             sem0, sem1, sem2, sem3):
    wid = lax.axis_index("s") * NC + lax.axis_index("c")
    base = wid * BPW

    pltpu.sync_copy(user_hbm.at[pl.ds(base, BPW)], uidx_v)
    pltpu.sync_copy(item_hbm.at[pl.ds(base, BPW)], iidx_v)

    c0 = pltpu.async_copy(uf_hbm.at[uidx_v], uf_v, sem0)
    c1 = pltpu.async_copy(if_hbm.at[iidx_v], if_v, sem1)
    c2 = pltpu.async_copy(ub_hbm.at[uidx_v], ub_v, sem2)
    c3 = pltpu.async_copy(ib_hbm.at[iidx_v], ib_v, sem3)
    c0.wait()
    c1.wait()
    c2.wait()
    c3.wait()

    iota = lax.iota(jnp.int32, L)

    def group(g, _):
        rows = g * L + iota
        acc = ub_v[pl.ds(g * L, L)] + ib_v[pl.ds(g * L, L)]
        for f in range(F):
            col = jnp.full((L,), f, jnp.int32)
            u = plsc.load_gather(uf_v, [rows, col])
            it = plsc.load_gather(if_v, [rows, col])
            acc = acc + u * it
        out_v[pl.ds(g * L, L)] = acc
        return 0

    lax.fori_loop(0, GROUPS, group, 0)

    pltpu.sync_copy(out_v, out_hbm.at[pl.ds(base, BPW)])


@jax.jit
def _mf(user, item, user_factors, item_factors, user_biases, item_biases):
    mesh = plsc.VectorSubcoreMesh(core_axis_name="c", subcore_axis_name="s")
    run = pl.kernel(
        _mf_body,
        out_type=jax.ShapeDtypeStruct((BATCH,), jnp.float32),
        mesh=mesh,
        compiler_params=pltpu.CompilerParams(
            needs_layout_passes=False, use_tc_tiling_on_sc=False),
        scratch_types=[
            pltpu.VMEM((BPW,), jnp.int32),
            pltpu.VMEM((BPW,), jnp.int32),
            pltpu.VMEM((BPW, F), jnp.float32),
            pltpu.VMEM((BPW, F), jnp.float32),
            pltpu.VMEM((BPW,), jnp.float32),
            pltpu.VMEM((BPW,), jnp.float32),
            pltpu.VMEM((BPW,), jnp.float32),
            pltpu.SemaphoreType.DMA,
            pltpu.SemaphoreType.DMA,
            pltpu.SemaphoreType.DMA,
            pltpu.SemaphoreType.DMA,
        ],
    )
    return run(user, item, user_factors, item_factors,
               user_biases.reshape(-1), item_biases.reshape(-1))


def kernel(user, item, user_factors, item_factors, user_biases, item_biases):
    return _mf(user, item, user_factors, item_factors, user_biases, item_biases)
